# baseline (speedup 1.0000x reference)
import functools

import jax
import jax.numpy as jnp
from jax import lax
from jax.experimental import pallas as pl
from jax.experimental.pallas import tpu as pltpu
from jax.experimental.pallas import tpu_sc as plsc

N = 100000
D = 128
E = 100000

NC = 2
NS = 16
NW = NC * NS
EP = 102400
EPW = EP // NW

NP = 102400
ZCH = NP // NS

_sc_mesh = plsc.VectorSubcoreMesh(core_axis_name="c", subcore_axis_name="s")


TAIL = E - (NW - 1) * EPW


@functools.partial(
    pl.kernel,
    mesh=_sc_mesh,
    out_type=jax.ShapeDtypeStruct((NC, NP), jnp.float32),
    scratch_types=[
        pltpu.VMEM((EPW,), jnp.int32),
        pltpu.VMEM((EPW,), jnp.float32),
        pltpu.VMEM((ZCH,), jnp.float32),
        pltpu.VMEM_SHARED((NP,), jnp.float32),
        pltpu.SemaphoreType.DMA,
        pltpu.SemaphoreType.DMA,
    ],
)
def _segment_sum_sc(ei_hbm, valp_hbm, zero_hbm, zi_hbm, out_hbm,
                    idx_v, val_v, stage_v, acc_sh, semA, semB):
    c = lax.axis_index("c")
    s = lax.axis_index("s")
    wid = c * NS + s
    base = wid * EPW

    a2 = pltpu.async_copy(valp_hbm.at[pl.ds(base, EPW)], val_v, semA)

    @pl.when(wid < NW - 1)
    def _():
        pltpu.sync_copy(ei_hbm.at[pl.ds(E + base, EPW)], idx_v)

    @pl.when(wid == NW - 1)
    def _():
        pltpu.sync_copy(ei_hbm.at[pl.ds(E + base, TAIL)],
                        idx_v.at[pl.ds(0, TAIL)])
        pltpu.sync_copy(zi_hbm, idx_v.at[pl.ds(TAIL, EPW - TAIL)])

    pltpu.sync_copy(zero_hbm.at[pl.ds(s * ZCH, ZCH)], stage_v)
    a3 = pltpu.async_copy(stage_v, acc_sh.at[pl.ds(s * ZCH, ZCH)], semB)

    a2.wait()
    a3.wait()
    plsc.subcore_barrier()

    pltpu.sync_copy(val_v, acc_sh.at[idx_v], add=True)

    plsc.subcore_barrier()

    pltpu.sync_copy(acc_sh.at[pl.ds(s * ZCH, ZCH)], stage_v)
    pltpu.sync_copy(stage_v, out_hbm.at[c, pl.ds(s * ZCH, ZCH)])


BN = 4096
M = BN // 128
G = NP // 128


def _gnn_tc_kernel(x_ref, wt_ref, b_ref, s_ref, o_ref):
    y = jnp.dot(x_ref[...], wt_ref[...], preferred_element_type=jnp.float32)
    y = jnp.maximum(y + b_ref[...], 0.0)
    s2 = s_ref[0] + s_ref[1]
    lane = lax.broadcasted_iota(jnp.int32, (1, D), 1)
    e0 = (lane == 0).astype(jnp.float32)
    adds = [
        lax.dot_general(s2[q:q + 1, :], e0, (((0,), (0,)), ((), ())),
                        preferred_element_type=jnp.float32)
        for q in range(M)
    ]
    o_ref[...] = y + jnp.concatenate(adds, axis=0)


def kernel(x, edge_index, W, b):
    ei_flat = edge_index.reshape(-1)
    valp = jnp.pad(x[:, 0], (0, EP - E))
    zeros = jnp.zeros((NP,), jnp.float32)
    zi = jnp.zeros((EPW - TAIL,), jnp.int32)
    s = _segment_sum_sc(ei_flat, valp, zeros, zi)
    s3 = s.reshape(NC, G, 128)

    wt = W.T
    b2 = b.reshape(1, D)
    return pl.pallas_call(
        _gnn_tc_kernel,
        grid=(NP // BN,),
        in_specs=[
            pl.BlockSpec((BN, D), lambda i: (i, 0)),
            pl.BlockSpec((D, D), lambda i: (0, 0)),
            pl.BlockSpec((1, D), lambda i: (0, 0)),
            pl.BlockSpec((NC, M, 128), lambda i: (0, i, 0)),
        ],
        out_specs=pl.BlockSpec((BN, D), lambda i: (i, 0)),
        out_shape=jax.ShapeDtypeStruct((N, D), jnp.float32),
    )(x, wt, b2, s3)


# speedup vs baseline: 1.0167x; 1.0167x over previous
import functools

import jax
import jax.numpy as jnp
from jax import lax
from jax.experimental import pallas as pl
from jax.experimental.pallas import tpu as pltpu
from jax.experimental.pallas import tpu_sc as plsc

N = 100000
D = 128
E = 100000

NC = 2
NS = 16
NW = NC * NS
EP = 102400
EPW = EP // NW

NP = 102400
ZCH = NP // NS

_sc_mesh = plsc.VectorSubcoreMesh(core_axis_name="c", subcore_axis_name="s")


TAIL = E - (NW - 1) * EPW


@functools.partial(
    pl.kernel,
    mesh=_sc_mesh,
    out_type=jax.ShapeDtypeStruct((NC, NP), jnp.float32),
    scratch_types=[
        pltpu.VMEM((EPW,), jnp.int32),
        pltpu.VMEM((EPW,), jnp.int32),
        pltpu.VMEM((EPW,), jnp.float32),
        pltpu.VMEM((ZCH,), jnp.float32),
        pltpu.VMEM_SHARED((NP,), jnp.float32),
        pltpu.SemaphoreType.DMA,
        pltpu.SemaphoreType.DMA,
    ],
)
def _segment_sum_sc(ei_hbm, xf_hbm, idx2c_hbm, zero_hbm, zi_hbm, out_hbm,
                    idx_v, idx2_v, val_v, stage_v, acc_sh, semA, semB):
    c = lax.axis_index("c")
    s = lax.axis_index("s")
    wid = c * NS + s
    base = wid * EPW

    a2 = pltpu.async_copy(idx2c_hbm.at[pl.ds(base, EPW)], idx2_v, semA)

    @pl.when(wid < NW - 1)
    def _():
        pltpu.sync_copy(ei_hbm.at[pl.ds(E + base, EPW)], idx_v)

    @pl.when(wid == NW - 1)
    def _():
        pltpu.sync_copy(ei_hbm.at[pl.ds(E + base, TAIL)],
                        idx_v.at[pl.ds(0, TAIL)])
        pltpu.sync_copy(zi_hbm, idx_v.at[pl.ds(TAIL, EPW - TAIL)])

    pltpu.sync_copy(zero_hbm.at[pl.ds(s * ZCH, ZCH)], stage_v)
    a3 = pltpu.async_copy(stage_v, acc_sh.at[pl.ds(s * ZCH, ZCH)], semB)

    a2.wait()
    pltpu.sync_copy(xf_hbm.at[idx2_v], val_v)

    @pl.when(wid == NW - 1)
    def _():
        pltpu.sync_copy(zero_hbm.at[pl.ds(0, EPW - TAIL)],
                        val_v.at[pl.ds(TAIL, EPW - TAIL)])

    a3.wait()
    plsc.subcore_barrier()

    pltpu.sync_copy(val_v, acc_sh.at[idx_v], add=True)

    plsc.subcore_barrier()

    pltpu.sync_copy(acc_sh.at[pl.ds(s * ZCH, ZCH)], stage_v)
    pltpu.sync_copy(stage_v, out_hbm.at[c, pl.ds(s * ZCH, ZCH)])


BN = 4096
M = BN // 128
G = NP // 128


def _gnn_tc_kernel(x_ref, wt_ref, b_ref, s_ref, o_ref):
    y = jnp.dot(x_ref[...], wt_ref[...], preferred_element_type=jnp.float32)
    y = jnp.maximum(y + b_ref[...], 0.0)
    s2 = s_ref[0] + s_ref[1]
    lane = lax.broadcasted_iota(jnp.int32, (1, D), 1)
    e0 = (lane == 0).astype(jnp.float32)
    adds = [
        lax.dot_general(s2[q:q + 1, :], e0, (((0,), (0,)), ((), ())),
                        preferred_element_type=jnp.float32)
        for q in range(M)
    ]
    o_ref[...] = y + jnp.concatenate(adds, axis=0)


def kernel(x, edge_index, W, b):
    ei_flat = edge_index.reshape(-1)
    x_flat = x.reshape(-1)
    ar = jnp.arange(EP, dtype=jnp.int32)
    idx2c = jnp.where(ar < E, ar * D, 0)
    zeros = jnp.zeros((NP,), jnp.float32)
    zi = jnp.zeros((EPW - TAIL,), jnp.int32)
    s = _segment_sum_sc(ei_flat, x_flat, idx2c, zeros, zi)
    s3 = s.reshape(NC, G, 128)

    wt = W.T
    b2 = b.reshape(1, D)
    return pl.pallas_call(
        _gnn_tc_kernel,
        grid=(NP // BN,),
        in_specs=[
            pl.BlockSpec((BN, D), lambda i: (i, 0)),
            pl.BlockSpec((D, D), lambda i: (0, 0)),
            pl.BlockSpec((1, D), lambda i: (0, 0)),
            pl.BlockSpec((NC, M, 128), lambda i: (0, i, 0)),
        ],
        out_specs=pl.BlockSpec((BN, D), lambda i: (i, 0)),
        out_shape=jax.ShapeDtypeStruct((N, D), jnp.float32),
    )(x, wt, b2, s3)
